# baseline (speedup 1.0000x reference)
import functools

import jax
import jax.numpy as jnp
from jax import lax
from jax.experimental import pallas as pl
from jax.experimental.pallas import tpu as pltpu
from jax.experimental.pallas import tpu_sc as plsc

F32 = jnp.float32

_NUM_FEAT = 100000
_EMB_DIM = 128
_BATCH = 16384
_FIELDS = 100
_MAX_NORM = 1.0

_NC, _NS, _L = 2, 16, 16
_NBUF = 2
_EPG = 2
_NW = _NC * _NS
_BPW = _BATCH // _NW
_DV = _EMB_DIM // _L


def _norm_body(t_ref, o_ref):
    t = t_ref[...]
    nsq = jnp.sum(t * t, axis=1, keepdims=True)
    norm = jnp.sqrt(nsq)
    scale = jnp.minimum(1.0, _MAX_NORM / jnp.maximum(norm, 1e-7))
    o_ref[...] = t * scale


def _normalize_table(table):
    rows_per_blk = 2000
    grid = _NUM_FEAT // rows_per_blk
    return pl.pallas_call(
        _norm_body,
        grid=(grid,),
        in_specs=[pl.BlockSpec((rows_per_blk, _EMB_DIM), lambda i: (i, 0))],
        out_specs=pl.BlockSpec((rows_per_blk, _EMB_DIM), lambda i: (i, 0)),
        out_shape=jax.ShapeDtypeStruct((_NUM_FEAT, _EMB_DIM), F32),
    )(table)


_MESH = plsc.VectorSubcoreMesh(core_axis_name="c", subcore_axis_name="s")


@functools.partial(
    pl.kernel,
    out_type=jax.ShapeDtypeStruct((_BATCH,), F32),
    mesh=_MESH,
    scratch_types=[
        pltpu.VMEM((_BPW, 128), jnp.int32),
        pltpu.VMEM((_EPG * _FIELDS, _EMB_DIM), F32),
        pltpu.VMEM((_EPG * _FIELDS, _EMB_DIM), F32),
        pltpu.VMEM((_BPW,), F32),
        pltpu.SemaphoreType.DMA,
        pltpu.SemaphoreType.DMA,
    ],
    compiler_params=pltpu.CompilerParams(needs_layout_passes=False),
)
def _fm_sc(tn_hbm, x_hbm, out_hbm, idx_v, buf0, buf1, out_v, sem0, sem1):
    wid = lax.axis_index("s") * _NC + lax.axis_index("c")
    base = wid * _BPW

    pltpu.sync_copy(x_hbm.at[pl.ds(base, _BPW)], idx_v)

    def start_gather(p, buf, sem):
        g = p * _EPG
        pltpu.async_copy(
            tn_hbm.at[idx_v.at[g, pl.ds(0, _FIELDS)]],
            buf.at[pl.ds(0, _FIELDS)], sem,
        )
        pltpu.async_copy(
            tn_hbm.at[idx_v.at[g + 1, pl.ds(0, _FIELDS)]],
            buf.at[pl.ds(_FIELDS, _FIELDS)], sem,
        )

    def wait_gather(p, buf, sem):
        g = p * _EPG
        pltpu.make_async_copy(
            tn_hbm.at[idx_v.at[g, pl.ds(0, _FIELDS)]],
            buf.at[pl.ds(0, _FIELDS)], sem,
        ).wait()
        pltpu.make_async_copy(
            tn_hbm.at[idx_v.at[g + 1, pl.ds(0, _FIELDS)]],
            buf.at[pl.ds(_FIELDS, _FIELDS)], sem,
        ).wait()

    def do_elem(g, buf, off, sem):
        def fbody(f, c):
            accs = list(c)
            t_acc = accs[_DV]
            for d in range(_DV):
                v = buf[off + f, pl.ds(_L * d, _L)]
                accs[d] = accs[d] + v
                t_acc = t_acc + v * v
            accs[_DV] = t_acc
            return tuple(accs)

        zero = jnp.zeros((_L,), F32)
        c = lax.fori_loop(0, _FIELDS, fbody, (zero,) * (_DV + 1), unroll=4)

        ssq = c[0] * c[0]
        for d in range(1, _DV):
            ssq = ssq + c[d] * c[d]
        tot = plsc.cumsum(0.5 * (ssq - c[_DV]))
        lane = lax.iota(jnp.int32, _L)
        plsc.store_scatter(
            out_v,
            [jnp.full((_L,), g, jnp.int32)],
            tot,
            mask=lane == (_L - 1),
        )

    bufs = (buf0, buf1)
    sems = (sem0, sem1)
    npairs = _BPW // _EPG
    for b in range(_NBUF):
        start_gather(b, bufs[b], sems[b])

    def do_pair(p, buf, sem):
        wait_gather(p, buf, sem)
        do_elem(p * _EPG, buf, 0, sem)
        do_elem(p * _EPG + 1, buf, _FIELDS, sem)
        @pl.when(p + _NBUF < npairs)
        def _():
            start_gather(p + _NBUF, buf, sem)

    def body(i, carry):
        p = i * _NBUF
        for b in range(_NBUF):
            do_pair(p + b, bufs[b], sems[b])
        return carry

    lax.fori_loop(0, npairs // _NBUF, body, 0)

    pltpu.sync_copy(out_v, out_hbm.at[pl.ds(base, _BPW)])


@jax.jit
def kernel(x, table):
    tn = _normalize_table(table)
    x_pad = jnp.pad(x, ((0, 0), (0, 128 - _FIELDS)))
    return _fm_sc(tn, x_pad).reshape(_BATCH, 1)


# speedup vs baseline: 1.0348x; 1.0348x over previous
import functools

import jax
import jax.numpy as jnp
from jax import lax
from jax.experimental import pallas as pl
from jax.experimental.pallas import tpu as pltpu
from jax.experimental.pallas import tpu_sc as plsc

F32 = jnp.float32

_NUM_FEAT = 100000
_EMB_DIM = 128
_BATCH = 16384
_FIELDS = 100
_MAX_NORM = 1.0

_NC, _NS, _L = 2, 16, 16
_NBUF = 4
_NW = _NC * _NS
_BPW = _BATCH // _NW
_DV = _EMB_DIM // _L


def _norm_body(t_ref, o_ref):
    t = t_ref[...]
    nsq = jnp.sum(t * t, axis=1, keepdims=True)
    norm = jnp.sqrt(nsq)
    scale = jnp.minimum(1.0, _MAX_NORM / jnp.maximum(norm, 1e-7))
    o_ref[...] = t * scale


def _normalize_table(table):
    rows_per_blk = 5000
    grid = _NUM_FEAT // rows_per_blk
    return pl.pallas_call(
        _norm_body,
        grid=(grid,),
        in_specs=[pl.BlockSpec((rows_per_blk, _EMB_DIM), lambda i: (i, 0))],
        out_specs=pl.BlockSpec((rows_per_blk, _EMB_DIM), lambda i: (i, 0)),
        out_shape=jax.ShapeDtypeStruct((_NUM_FEAT, _EMB_DIM), F32),
    )(table)


_MESH = plsc.VectorSubcoreMesh(core_axis_name="c", subcore_axis_name="s")


@functools.partial(
    pl.kernel,
    out_type=jax.ShapeDtypeStruct((_BATCH,), F32),
    mesh=_MESH,
    scratch_types=[
        pltpu.VMEM((_BPW, 128), jnp.int32),
        pltpu.VMEM((_FIELDS, _EMB_DIM), F32),
        pltpu.VMEM((_FIELDS, _EMB_DIM), F32),
        pltpu.VMEM((_FIELDS, _EMB_DIM), F32),
        pltpu.VMEM((_FIELDS, _EMB_DIM), F32),
        pltpu.VMEM((_BPW,), F32),
        pltpu.SemaphoreType.DMA,
        pltpu.SemaphoreType.DMA,
        pltpu.SemaphoreType.DMA,
        pltpu.SemaphoreType.DMA,
    ],
    compiler_params=pltpu.CompilerParams(needs_layout_passes=False),
)
def _fm_sc(tn_hbm, x_hbm, out_hbm, idx_v, buf0, buf1, buf2, buf3, out_v,
           sem0, sem1, sem2, sem3):
    wid = lax.axis_index("s") * _NC + lax.axis_index("c")
    base = wid * _BPW

    pltpu.sync_copy(x_hbm.at[pl.ds(base, _BPW)], idx_v)

    def start_gather(g, buf, sem):
        pltpu.async_copy(tn_hbm.at[idx_v.at[g, pl.ds(0, _FIELDS)]], buf, sem)

    def wait_gather(g, buf, sem):
        pltpu.make_async_copy(
            tn_hbm.at[idx_v.at[g, pl.ds(0, _FIELDS)]], buf, sem
        ).wait()

    def do_elem(g, buf, sem):
        wait_gather(g, buf, sem)

        def fbody(f, c):
            accs = list(c)
            t_acc = accs[_DV]
            for d in range(_DV):
                v = buf[f, pl.ds(_L * d, _L)]
                accs[d] = accs[d] + v
                t_acc = t_acc + v * v
            accs[_DV] = t_acc
            return tuple(accs)

        zero = jnp.zeros((_L,), F32)
        c = lax.fori_loop(0, _FIELDS, fbody, (zero,) * (_DV + 1), unroll=4)

        ssq = c[0] * c[0]
        for d in range(1, _DV):
            ssq = ssq + c[d] * c[d]
        tot = plsc.cumsum(0.5 * (ssq - c[_DV]))
        lane = lax.iota(jnp.int32, _L)
        plsc.store_scatter(
            out_v,
            [jnp.full((_L,), g, jnp.int32)],
            tot,
            mask=lane == (_L - 1),
        )

        @pl.when(g + _NBUF < _BPW)
        def _():
            start_gather(g + _NBUF, buf, sem)

    bufs = (buf0, buf1, buf2, buf3)
    sems = (sem0, sem1, sem2, sem3)
    for b in range(_NBUF):
        start_gather(b, bufs[b], sems[b])

    def body(i, carry):
        g = i * _NBUF
        for b in range(_NBUF):
            do_elem(g + b, bufs[b], sems[b])
        return carry

    lax.fori_loop(0, _BPW // _NBUF, body, 0)

    pltpu.sync_copy(out_v, out_hbm.at[pl.ds(base, _BPW)])


@jax.jit
def kernel(x, table):
    tn = _normalize_table(table)
    x_pad = jnp.pad(x, ((0, 0), (0, 128 - _FIELDS)))
    return _fm_sc(tn, x_pad).reshape(_BATCH, 1)


# speedup vs baseline: 1.0424x; 1.0074x over previous
import functools

import jax
import jax.numpy as jnp
from jax import lax
from jax.experimental import pallas as pl
from jax.experimental.pallas import tpu as pltpu
from jax.experimental.pallas import tpu_sc as plsc

F32 = jnp.float32

_NUM_FEAT = 100000
_EMB_DIM = 128
_BATCH = 16384
_FIELDS = 100
_MAX_NORM = 1.0

_NC, _NS, _L = 2, 16, 16
_NBUF = 4
_NW = _NC * _NS
_BPW = _BATCH // _NW
_DV = _EMB_DIM // _L


def _norm_body(t_ref, o_ref):
    t = t_ref[...]
    nsq = jnp.sum(t * t, axis=1, keepdims=True)
    norm = jnp.sqrt(nsq)
    scale = jnp.minimum(1.0, _MAX_NORM / jnp.maximum(norm, 1e-7))
    o_ref[...] = t * scale


def _normalize_table(table):
    rows_per_blk = 10000
    grid = _NUM_FEAT // rows_per_blk
    return pl.pallas_call(
        _norm_body,
        grid=(grid,),
        in_specs=[pl.BlockSpec((rows_per_blk, _EMB_DIM), lambda i: (i, 0))],
        out_specs=pl.BlockSpec((rows_per_blk, _EMB_DIM), lambda i: (i, 0)),
        out_shape=jax.ShapeDtypeStruct((_NUM_FEAT, _EMB_DIM), F32),
    )(table)


_MESH = plsc.VectorSubcoreMesh(core_axis_name="c", subcore_axis_name="s")


@functools.partial(
    pl.kernel,
    out_type=jax.ShapeDtypeStruct((_BATCH,), F32),
    mesh=_MESH,
    scratch_types=[
        pltpu.VMEM((_BPW, 128), jnp.int32),
        pltpu.VMEM((_FIELDS, _EMB_DIM), F32),
        pltpu.VMEM((_FIELDS, _EMB_DIM), F32),
        pltpu.VMEM((_FIELDS, _EMB_DIM), F32),
        pltpu.VMEM((_FIELDS, _EMB_DIM), F32),
        pltpu.VMEM((_BPW,), F32),
        pltpu.SemaphoreType.DMA,
        pltpu.SemaphoreType.DMA,
        pltpu.SemaphoreType.DMA,
        pltpu.SemaphoreType.DMA,
    ],
    compiler_params=pltpu.CompilerParams(needs_layout_passes=False),
)
def _fm_sc(tn_hbm, x_hbm, out_hbm, idx_v, buf0, buf1, buf2, buf3, out_v,
           sem0, sem1, sem2, sem3):
    wid = lax.axis_index("s") * _NC + lax.axis_index("c")
    base = wid * _BPW

    pltpu.sync_copy(x_hbm.at[pl.ds(base, _BPW)], idx_v)

    def start_gather(g, buf, sem):
        pltpu.async_copy(tn_hbm.at[idx_v.at[g, pl.ds(0, _FIELDS)]], buf, sem)

    def wait_gather(g, buf, sem):
        pltpu.make_async_copy(
            tn_hbm.at[idx_v.at[g, pl.ds(0, _FIELDS)]], buf, sem
        ).wait()

    def do_elem(g, buf, sem):
        wait_gather(g, buf, sem)

        def fbody(f, c):
            accs = list(c)
            t_acc = accs[_DV]
            for d in range(_DV):
                v = buf[f, pl.ds(_L * d, _L)]
                accs[d] = accs[d] + v
                t_acc = t_acc + v * v
            accs[_DV] = t_acc
            return tuple(accs)

        zero = jnp.zeros((_L,), F32)
        c = lax.fori_loop(0, _FIELDS, fbody, (zero,) * (_DV + 1), unroll=4)

        ssq = c[0] * c[0]
        for d in range(1, _DV):
            ssq = ssq + c[d] * c[d]
        tot = plsc.cumsum(0.5 * (ssq - c[_DV]))
        lane = lax.iota(jnp.int32, _L)
        plsc.store_scatter(
            out_v,
            [jnp.full((_L,), g, jnp.int32)],
            tot,
            mask=lane == (_L - 1),
        )

        @pl.when(g + _NBUF < _BPW)
        def _():
            start_gather(g + _NBUF, buf, sem)

    bufs = (buf0, buf1, buf2, buf3)
    sems = (sem0, sem1, sem2, sem3)
    for b in range(_NBUF):
        start_gather(b, bufs[b], sems[b])

    def body(i, carry):
        g = i * _NBUF
        for b in range(_NBUF):
            do_elem(g + b, bufs[b], sems[b])
        return carry

    lax.fori_loop(0, _BPW // _NBUF, body, 0)

    pltpu.sync_copy(out_v, out_hbm.at[pl.ds(base, _BPW)])


@jax.jit
def kernel(x, table):
    tn = _normalize_table(table)
    x_pad = jnp.pad(x, ((0, 0), (0, 128 - _FIELDS)))
    return _fm_sc(tn, x_pad).reshape(_BATCH, 1)
